# Initial kernel scaffold; baseline (speedup 1.0000x reference)
import jax
import jax.numpy as jnp
from jax.experimental import pallas as pl


def kernel(x, edge_index, batch, c1W1, c1b1, c1W2, c1b2, c2W1, c2b1, c2W2, c2b2, c3W1, c3b1, c3W2, c3b2, c4W1, c4b1, c4W2, c4b2, n1W, n1b, n2W, n2b, n3W, n3b, n4W, n4b):
    raise NotImplementedError("write your pallas kernel here")


# speedup vs baseline: 10.1765x; 10.1765x over previous
import jax
import jax.numpy as jnp
from jax.experimental import pallas as pl

_B, _S, _K = 256, 192, 4
_N = _B * _S


def _leaky(v):
    return jnp.where(v >= 0, v, v * 0.01)


def _dot_split(parts, W):
    off = 0
    acc = None
    for p_ in parts:
        w = W[off:off + p_.shape[1], :]
        t = jnp.dot(p_, w, preferred_element_type=jnp.float32)
        acc = t if acc is None else acc + t
        off += p_.shape[1]
    return acc


def _knn_gather(xb):
    pos = xb[:, 0:3]
    post = pos.T
    d2 = (pos[:, 0:1] - post[0:1, :]) ** 2
    d2 = d2 + (pos[:, 1:2] - post[1:2, :]) ** 2
    d2 = d2 + (pos[:, 2:3] - post[2:3, :]) ** 2
    col = jax.lax.broadcasted_iota(jnp.int32, (_S, _S), 1)
    row = jax.lax.broadcasted_iota(jnp.int32, (_S, _S), 0)
    d2 = jnp.where(row == col, d2 + 1e9, d2)
    xjs = []
    for _ in range(_K):
        mval = jnp.min(d2, axis=1, keepdims=True)
        idxk = jnp.min(jnp.where(d2 == mval, col, _S), axis=1, keepdims=True)
        sel = col == idxk
        onehot = sel.astype(xb.dtype)
        xjs.append(jnp.dot(onehot, xb, preferred_element_type=jnp.float32))
        d2 = jnp.where(sel, 1e9, d2)
    return xjs


def _edge_conv(xb, W1, b1, W2, b2):
    xjs = _knn_gather(xb)
    acc = None
    for xj in xjs:
        h = _leaky(_dot_split([xb, xj - xb], W1) + b1)
        h = _leaky(jnp.dot(h, W2, preferred_element_type=jnp.float32) + b2)
        acc = h if acc is None else acc + h
    return acc


def _body(x_ref, c1W1, c1b1, c1W2, c1b2, c2W1, c2b1, c2W2, c2b2,
          c3W1, c3b1, c3W2, c3b2, c4W1, c4b1, c4W2, c4b2,
          n1W, n1b, n2W, n2b, n3W, n3b, n4W, n4b, out_ref):
    xb = x_ref[...]
    a = _edge_conv(xb, c1W1[...], c1b1[...], c1W2[...], c1b2[...])
    b = _edge_conv(a, c2W1[...], c2b1[...], c2W2[...], c2b2[...])
    c = _edge_conv(b, c3W1[...], c3b1[...], c3W2[...], c3b2[...])
    d = _edge_conv(c, c4W1[...], c4b1[...], c4W2[...], c4b2[...])
    h = _leaky(_dot_split([xb, a, b, c, d], n1W[...]) + n1b[...])
    h = jnp.dot(h, n2W[...], preferred_element_type=jnp.float32) + n2b[...]
    mx = jnp.max(h, axis=0, keepdims=True)
    mn = jnp.min(h, axis=0, keepdims=True)
    sm = jnp.sum(h, axis=0, keepdims=True)
    mean = sm / jnp.float32(_S)
    p = _dot_split([_leaky(mx), _leaky(mn), _leaky(sm), _leaky(mean)],
                   n3W[...])
    p = _leaky(p + n3b[...])
    o = jnp.dot(p, n4W[...], preferred_element_type=jnp.float32) + n4b[...]
    out_ref[...] = o.reshape(1, 1, 1)


def kernel(x, edge_index, batch, c1W1, c1b1, c1W2, c1b2, c2W1, c2b1, c2W2,
           c2b2, c3W1, c3b1, c3W2, c3b2, c4W1, c4b1, c4W2, c4b2, n1W, n1b,
           n2W, n2b, n3W, n3b, n4W, n4b):
    del edge_index, batch
    weights = [c1W1, c1b1, c1W2, c1b2, c2W1, c2b1, c2W2, c2b2,
               c3W1, c3b1, c3W2, c3b2, c4W1, c4b1, c4W2, c4b2,
               n1W, n1b, n2W, n2b, n3W, n3b, n4W, n4b]
    weights = [w.reshape(1, -1) if w.ndim == 1 else w for w in weights]

    def _wspec(w):
        return pl.BlockSpec(w.shape, lambda g: (0,) * w.ndim)

    out = pl.pallas_call(
        _body,
        grid=(_B,),
        in_specs=[pl.BlockSpec((_S, 4), lambda g: (g, 0))]
                 + [_wspec(w) for w in weights],
        out_specs=pl.BlockSpec((1, 1, 1), lambda g: (g, 0, 0)),
        out_shape=jax.ShapeDtypeStruct((_B, 1, 1), jnp.float32),
    )(x, *weights)
    return out.reshape(_B, 1)


# speedup vs baseline: 10.9493x; 1.0759x over previous
import jax
import jax.numpy as jnp
from jax.experimental import pallas as pl

_B, _S, _K = 256, 192, 4
_N = _B * _S


def _leaky(v):
    return jnp.where(v >= 0, v, v * 0.01)


def _dot_split(parts, W):
    off = 0
    acc = None
    for p_ in parts:
        w = W[off:off + p_.shape[1], :]
        t = jnp.dot(p_, w, preferred_element_type=jnp.float32)
        acc = t if acc is None else acc + t
        off += p_.shape[1]
    return acc


def _knn_gather(xb):
    pos = xb[:, 0:3]
    post = pos.T
    d2 = (pos[:, 0:1] - post[0:1, :]) ** 2
    d2 = d2 + (pos[:, 1:2] - post[1:2, :]) ** 2
    d2 = d2 + (pos[:, 2:3] - post[2:3, :]) ** 2
    col = jax.lax.broadcasted_iota(jnp.int32, (_S, _S), 1)
    row = jax.lax.broadcasted_iota(jnp.int32, (_S, _S), 0)
    d2 = jnp.where(row == col, d2 + 1e9, d2)
    xjs = []
    for _ in range(_K):
        mval = jnp.min(d2, axis=1, keepdims=True)
        idxk = jnp.min(jnp.where(d2 == mval, col, _S), axis=1, keepdims=True)
        sel = col == idxk
        onehot = sel.astype(xb.dtype)
        xjs.append(jnp.dot(onehot, xb, preferred_element_type=jnp.float32,
                           precision=jax.lax.Precision.HIGHEST))
        d2 = jnp.where(sel, 1e9, d2)
    return xjs


def _edge_conv(xb, W1, b1, W2, b2):
    xjs = _knn_gather(xb)
    acc = None
    for xj in xjs:
        m = jnp.concatenate([xb, xj - xb], axis=1)
        h = _leaky(jnp.dot(m, W1, preferred_element_type=jnp.float32) + b1)
        h = _leaky(jnp.dot(h, W2, preferred_element_type=jnp.float32) + b2)
        acc = h if acc is None else acc + h
    return acc


_G = 4


def _body(x_ref, c1W1, c1b1, c1W2, c1b2, c2W1, c2b1, c2W2, c2b2,
          c3W1, c3b1, c3W2, c3b2, c4W1, c4b1, c4W2, c4b2,
          n1W, n1b, n2W, n2b, n3W, n3b, n4W, n4b, out_ref):
    for i in range(_G):
        xb = x_ref[i * _S:(i + 1) * _S, :]
        a = _edge_conv(xb, c1W1[...], c1b1[...], c1W2[...], c1b2[...])
        b = _edge_conv(a, c2W1[...], c2b1[...], c2W2[...], c2b2[...])
        c = _edge_conv(b, c3W1[...], c3b1[...], c3W2[...], c3b2[...])
        d = _edge_conv(c, c4W1[...], c4b1[...], c4W2[...], c4b2[...])
        h = _leaky(_dot_split([xb, a, b, c, d], n1W[...]) + n1b[...])
        h = jnp.dot(h, n2W[...], preferred_element_type=jnp.float32) + n2b[...]
        mx = jnp.max(h, axis=0, keepdims=True)
        mn = jnp.min(h, axis=0, keepdims=True)
        sm = jnp.sum(h, axis=0, keepdims=True)
        mean = sm / jnp.float32(_S)
        p = _dot_split([_leaky(mx), _leaky(mn), _leaky(sm), _leaky(mean)],
                       n3W[...])
        p = _leaky(p + n3b[...])
        o = jnp.dot(p, n4W[...], preferred_element_type=jnp.float32) + n4b[...]
        out_ref[i, :, :] = o.reshape(1, 1)


def kernel(x, edge_index, batch, c1W1, c1b1, c1W2, c1b2, c2W1, c2b1, c2W2,
           c2b2, c3W1, c3b1, c3W2, c3b2, c4W1, c4b1, c4W2, c4b2, n1W, n1b,
           n2W, n2b, n3W, n3b, n4W, n4b):
    del edge_index, batch
    weights = [c1W1, c1b1, c1W2, c1b2, c2W1, c2b1, c2W2, c2b2,
               c3W1, c3b1, c3W2, c3b2, c4W1, c4b1, c4W2, c4b2,
               n1W, n1b, n2W, n2b, n3W, n3b, n4W, n4b]
    weights = [w.reshape(1, -1) if w.ndim == 1 else w for w in weights]

    def _wspec(w):
        return pl.BlockSpec(w.shape, lambda g: (0,) * w.ndim)

    out = pl.pallas_call(
        _body,
        grid=(_B // _G,),
        in_specs=[pl.BlockSpec((_G * _S, 4), lambda g: (g, 0))]
                 + [_wspec(w) for w in weights],
        out_specs=pl.BlockSpec((_G, 1, 1), lambda g: (g, 0, 0)),
        out_shape=jax.ShapeDtypeStruct((_B, 1, 1), jnp.float32),
    )(x, *weights)
    return out.reshape(_B, 1)


# speedup vs baseline: 11.9741x; 1.0936x over previous
import jax
import jax.numpy as jnp
from jax.experimental import pallas as pl
from jax.experimental.pallas import tpu as pltpu

_B, _S, _K = 256, 192, 4
_N = _B * _S


def _leaky(v):
    return jnp.where(v >= 0, v, v * 0.01)


def _dot_split(parts, W):
    off = 0
    acc = None
    for p_ in parts:
        w = W[off:off + p_.shape[1], :]
        t = jnp.dot(p_, w, preferred_element_type=jnp.float32)
        acc = t if acc is None else acc + t
        off += p_.shape[1]
    return acc


def _exact_gather(onehot, hi, mid, lo, s1, s2):
    d = hi.shape[1]
    s1[:, 0:d] = jnp.dot(onehot, hi, preferred_element_type=jnp.float32)
    s2[:, 0:d] = jnp.dot(onehot, mid, preferred_element_type=jnp.float32)
    g3 = jnp.dot(onehot, lo, preferred_element_type=jnp.float32)
    return (s1[:, 0:d] + s2[:, 0:d]) + g3


def _bf16x3(x):
    hi = x.astype(jnp.bfloat16)
    r1 = x - hi.astype(jnp.float32)
    mid = r1.astype(jnp.bfloat16)
    r2 = r1 - mid.astype(jnp.float32)
    lo = r2.astype(jnp.bfloat16)
    return hi, mid, lo


def _knn_gather(xb, spairs):
    pos = xb[:, 0:3]
    post = pos.T
    d2 = (pos[:, 0:1] - post[0:1, :]) ** 2
    d2 = d2 + (pos[:, 1:2] - post[1:2, :]) ** 2
    d2 = d2 + (pos[:, 2:3] - post[2:3, :]) ** 2
    col = jax.lax.broadcasted_iota(jnp.int32, (_S, _S), 1)
    row = jax.lax.broadcasted_iota(jnp.int32, (_S, _S), 0)
    d2 = jnp.where(row == col, d2 + 1e9, d2)
    hi, mid, lo = _bf16x3(xb)
    xjs = []
    for k in range(_K):
        mval = jnp.min(d2, axis=1, keepdims=True)
        idxk = jnp.min(jnp.where(d2 == mval, col, _S), axis=1, keepdims=True)
        sel = col == idxk
        onehot = sel.astype(jnp.bfloat16)
        s1, s2 = spairs[k]
        xjs.append(_exact_gather(onehot, hi, mid, lo, s1, s2))
        d2 = jnp.where(sel, 1e9, d2)
    return xjs


def _edge_conv(xb, W1, b1, W2, b2, spairs):
    xjs = _knn_gather(xb, spairs)
    acc = None
    for xj in xjs:
        m = jnp.concatenate([xb, xj - xb], axis=1)
        h = _leaky(jnp.dot(m, W1, preferred_element_type=jnp.float32) + b1)
        h = _leaky(jnp.dot(h, W2, preferred_element_type=jnp.float32) + b2)
        acc = h if acc is None else acc + h
    return acc


_G = 4


def _body(x_ref, c1W1, c1b1, c1W2, c1b2, c2W1, c2b1, c2W2, c2b2,
          c3W1, c3b1, c3W2, c3b2, c4W1, c4b1, c4W2, c4b2,
          n1W, n1b, n2W, n2b, n3W, n3b, n4W, n4b, out_ref, *scratch):
    for i in range(_G):
        spairs = [(scratch[(i * _K + k) * 2], scratch[(i * _K + k) * 2 + 1])
                  for k in range(_K)]
        xb = x_ref[i * _S:(i + 1) * _S, :]
        a = _edge_conv(xb, c1W1[...], c1b1[...], c1W2[...], c1b2[...], spairs)
        b = _edge_conv(a, c2W1[...], c2b1[...], c2W2[...], c2b2[...], spairs)
        c = _edge_conv(b, c3W1[...], c3b1[...], c3W2[...], c3b2[...], spairs)
        d = _edge_conv(c, c4W1[...], c4b1[...], c4W2[...], c4b2[...], spairs)
        h = _leaky(_dot_split([xb, a, b, c, d], n1W[...]) + n1b[...])
        h = jnp.dot(h, n2W[...], preferred_element_type=jnp.float32) + n2b[...]
        mx = jnp.max(h, axis=0, keepdims=True)
        mn = jnp.min(h, axis=0, keepdims=True)
        sm = jnp.sum(h, axis=0, keepdims=True)
        mean = sm / jnp.float32(_S)
        p = _dot_split([_leaky(mx), _leaky(mn), _leaky(sm), _leaky(mean)],
                       n3W[...])
        p = _leaky(p + n3b[...])
        o = jnp.dot(p, n4W[...], preferred_element_type=jnp.float32) + n4b[...]
        out_ref[i, :, :] = o.reshape(1, 1)


def kernel(x, edge_index, batch, c1W1, c1b1, c1W2, c1b2, c2W1, c2b1, c2W2,
           c2b2, c3W1, c3b1, c3W2, c3b2, c4W1, c4b1, c4W2, c4b2, n1W, n1b,
           n2W, n2b, n3W, n3b, n4W, n4b):
    del edge_index, batch
    weights = [c1W1, c1b1, c1W2, c1b2, c2W1, c2b1, c2W2, c2b2,
               c3W1, c3b1, c3W2, c3b2, c4W1, c4b1, c4W2, c4b2,
               n1W, n1b, n2W, n2b, n3W, n3b, n4W, n4b]
    weights = [w.reshape(1, -1) if w.ndim == 1 else w for w in weights]

    def _wspec(w):
        return pl.BlockSpec(w.shape, lambda g: (0,) * w.ndim)

    out = pl.pallas_call(
        _body,
        grid=(_B // _G,),
        in_specs=[pl.BlockSpec((_G * _S, 4), lambda g: (g, 0))]
                 + [_wspec(w) for w in weights],
        out_specs=pl.BlockSpec((_G, 1, 1), lambda g: (g, 0, 0)),
        out_shape=jax.ShapeDtypeStruct((_B, 1, 1), jnp.float32),
        scratch_shapes=[pltpu.VMEM((_S, 64), jnp.float32)
                        for _ in range(_G * _K * 2)],
    )(x, *weights)
    return out.reshape(_B, 1)


# speedup vs baseline: 13.0837x; 1.0927x over previous
import jax
import jax.numpy as jnp
from jax.experimental import pallas as pl
from jax.experimental.pallas import tpu as pltpu

_B, _S, _K = 256, 192, 4
_N = _B * _S


def _leaky(v):
    return jnp.where(v >= 0, v, v * 0.01)


def _dot_split(parts, W):
    off = 0
    acc = None
    for p_ in parts:
        w = W[off:off + p_.shape[1], :]
        t = jnp.dot(p_, w, preferred_element_type=jnp.float32)
        acc = t if acc is None else acc + t
        off += p_.shape[1]
    return acc


def _bf16x3(x):
    hi = x.astype(jnp.bfloat16)
    r1 = x - hi.astype(jnp.float32)
    mid = r1.astype(jnp.bfloat16)
    r2 = r1 - mid.astype(jnp.float32)
    lo = r2.astype(jnp.bfloat16)
    return hi, mid, lo


def _exact_gather(onehot, x, s1, s2):
    hi, mid, lo = _bf16x3(x)
    d = x.shape[1]
    s1[:, 0:d] = jnp.dot(onehot, hi, preferred_element_type=jnp.float32)
    s2[:, 0:d] = jnp.dot(onehot, mid, preferred_element_type=jnp.float32)
    g3 = jnp.dot(onehot, lo, preferred_element_type=jnp.float32)
    return (s1[:, 0:d] + s2[:, 0:d]) + g3


def _knn_onehots(xb):
    pos = xb[:, 0:3]
    post = pos.T
    d2 = (pos[:, 0:1] - post[0:1, :]) ** 2
    d2 = d2 + (pos[:, 1:2] - post[1:2, :]) ** 2
    d2 = d2 + (pos[:, 2:3] - post[2:3, :]) ** 2
    colf = jax.lax.broadcasted_iota(jnp.int32, (_S, _S), 1).astype(jnp.float32)
    rowf = jax.lax.broadcasted_iota(jnp.int32, (_S, _S), 0).astype(jnp.float32)
    d2 = jnp.where(rowf == colf, d2 + 1e9, d2)
    ohs = []
    for _ in range(_K):
        mval = jnp.min(d2, axis=1, keepdims=True)
        idxk = jnp.min(jnp.where(d2 == mval, colf, jnp.float32(_S)),
                       axis=1, keepdims=True)
        sel = colf == idxk
        ohs.append(sel.astype(jnp.bfloat16))
        d2 = jnp.where(sel, 1e9, d2)
    return jnp.concatenate(ohs, axis=0)


def _edge_conv(xb, W1, b1, W2, b2, s1, s2):
    oh = _knn_onehots(xb)
    xj = _exact_gather(oh, xb, s1, s2)
    xi = jnp.concatenate([xb, xb, xb, xb], axis=0)
    m = jnp.concatenate([xi, xj - xi], axis=1)
    h = _leaky(jnp.dot(m, W1, preferred_element_type=jnp.float32) + b1)
    h = _leaky(jnp.dot(h, W2, preferred_element_type=jnp.float32) + b2)
    return ((h[0:_S] + h[_S:2 * _S]) + h[2 * _S:3 * _S]) + h[3 * _S:4 * _S]


_G = 4


def _body(x_ref, c1W1, c1b1, c1W2, c1b2, c2W1, c2b1, c2W2, c2b2,
          c3W1, c3b1, c3W2, c3b2, c4W1, c4b1, c4W2, c4b2,
          n1W, n1b, n2W, n2b, n3W, n3b, n4W, n4b, out_ref, *scratch):
    for i in range(_G):
        s1, s2 = scratch[2 * i], scratch[2 * i + 1]
        xb = x_ref[i * _S:(i + 1) * _S, :]
        a = _edge_conv(xb, c1W1[...], c1b1[...], c1W2[...], c1b2[...], s1, s2)
        b = _edge_conv(a, c2W1[...], c2b1[...], c2W2[...], c2b2[...], s1, s2)
        c = _edge_conv(b, c3W1[...], c3b1[...], c3W2[...], c3b2[...], s1, s2)
        d = _edge_conv(c, c4W1[...], c4b1[...], c4W2[...], c4b2[...], s1, s2)
        h = _leaky(_dot_split([xb, a, b, c, d], n1W[...]) + n1b[...])
        h = jnp.dot(h, n2W[...], preferred_element_type=jnp.float32) + n2b[...]
        mx = jnp.max(h, axis=0, keepdims=True)
        mn = jnp.min(h, axis=0, keepdims=True)
        sm = jnp.sum(h, axis=0, keepdims=True)
        mean = sm / jnp.float32(_S)
        p = _dot_split([_leaky(mx), _leaky(mn), _leaky(sm), _leaky(mean)],
                       n3W[...])
        p = _leaky(p + n3b[...])
        o = jnp.dot(p, n4W[...], preferred_element_type=jnp.float32) + n4b[...]
        out_ref[i, :, :] = o.reshape(1, 1)


def kernel(x, edge_index, batch, c1W1, c1b1, c1W2, c1b2, c2W1, c2b1, c2W2,
           c2b2, c3W1, c3b1, c3W2, c3b2, c4W1, c4b1, c4W2, c4b2, n1W, n1b,
           n2W, n2b, n3W, n3b, n4W, n4b):
    del edge_index, batch
    weights = [c1W1, c1b1, c1W2, c1b2, c2W1, c2b1, c2W2, c2b2,
               c3W1, c3b1, c3W2, c3b2, c4W1, c4b1, c4W2, c4b2,
               n1W, n1b, n2W, n2b, n3W, n3b, n4W, n4b]
    weights = [w.reshape(1, -1) if w.ndim == 1 else w for w in weights]

    def _wspec(w):
        return pl.BlockSpec(w.shape, lambda g: (0,) * w.ndim)

    out = pl.pallas_call(
        _body,
        grid=(_B // _G,),
        in_specs=[pl.BlockSpec((_G * _S, 4), lambda g: (g, 0))]
                 + [_wspec(w) for w in weights],
        out_specs=pl.BlockSpec((_G, 1, 1), lambda g: (g, 0, 0)),
        out_shape=jax.ShapeDtypeStruct((_B, 1, 1), jnp.float32),
        scratch_shapes=[pltpu.VMEM((_K * _S, 64), jnp.float32)
                        for _ in range(_G * 2)],
    )(x, *weights)
    return out.reshape(_B, 1)
